# Initial kernel scaffold; baseline (speedup 1.0000x reference)
import jax
import jax.numpy as jnp
from jax.experimental import pallas as pl


def kernel(node_feats, edge_feats, edge_index, Wn, bn, We1, be1, We2, be2, Wo, bo, Wp, bp):
    raise NotImplementedError("write your pallas kernel here")


# speedup vs baseline: 2.3454x; 2.3454x over previous
import functools

import jax
import jax.numpy as jnp
from jax import lax
from jax.experimental import pallas as pl
from jax.experimental.pallas import tpu as pltpu
from jax.experimental.pallas import tpu_sc as plsc

N = 10000
E = 320000
D = 128
DE = 16
H = 64
MAX_NEIGHBORS = 32.0

_NTILES = 32
_C = 128
_NCHUNK = E // _C
_ROWS_PT = 632
_NPAD = 16 * _ROWS_PT


def _ssp(x):
    return jax.nn.softplus(x) - jnp.log(2.0)


def _node_proj_body(x_ref, w_ref, b_ref, o_ref):
    o_ref[...] = (
        jnp.dot(x_ref[...], w_ref[...], preferred_element_type=jnp.float32)
        + b_ref[...]
    )


def _node_proj(x, w, b):
    bn = 2000
    return pl.pallas_call(
        _node_proj_body,
        grid=(N // bn,),
        in_specs=[
            pl.BlockSpec((bn, D), lambda i: (i, 0)),
            pl.BlockSpec((D, H), lambda i: (0, 0)),
            pl.BlockSpec((1, H), lambda i: (0, 0)),
        ],
        out_specs=pl.BlockSpec((bn, H), lambda i: (i, 0)),
        out_shape=jax.ShapeDtypeStruct((N, H), jnp.float32),
    )(x, w, b.reshape(1, H))


def _edge_mlp_body(ef_ref, w1_ref, b1_ref, w2_ref, b2_ref, o_ref):
    h = (
        jnp.dot(ef_ref[...], w1_ref[...], preferred_element_type=jnp.float32)
        + b1_ref[...]
    )
    h = _ssp(h)
    h = jnp.dot(h, w2_ref[...], preferred_element_type=jnp.float32) + b2_ref[...]
    o_ref[...] = _ssp(h)


def _edge_mlp(ef, w1, b1, w2, b2):
    be = 8000
    return pl.pallas_call(
        _edge_mlp_body,
        grid=(E // be,),
        in_specs=[
            pl.BlockSpec((be, DE), lambda i: (i, 0)),
            pl.BlockSpec((DE, H), lambda i: (0, 0)),
            pl.BlockSpec((1, H), lambda i: (0, 0)),
            pl.BlockSpec((H, H), lambda i: (0, 0)),
            pl.BlockSpec((1, H), lambda i: (0, 0)),
        ],
        out_specs=pl.BlockSpec((be, H), lambda i: (i, 0)),
        out_shape=jax.ShapeDtypeStruct((E, H), jnp.float32),
    )(ef, w1, b1.reshape(1, H), w2, b2.reshape(1, H))


def _final_body(p_ref, wo_ref, bo_ref, wp_ref, bp_ref, o_ref):
    h = p_ref[0] + p_ref[1]
    h = _ssp(
        jnp.dot(h, wo_ref[...], preferred_element_type=jnp.float32) + bo_ref[...]
    ) * (1.0 / MAX_NEIGHBORS)
    o_ref[...] = (
        jnp.dot(h, wp_ref[...], preferred_element_type=jnp.float32) + bp_ref[...]
    )


def _final(partials, wo, bo, wp, bp):
    bn = 2000
    return pl.pallas_call(
        _final_body,
        grid=(N // bn,),
        in_specs=[
            pl.BlockSpec((2, bn, H), lambda i: (0, i, 0)),
            pl.BlockSpec((H, D), lambda i: (0, 0)),
            pl.BlockSpec((1, D), lambda i: (0, 0)),
            pl.BlockSpec((D, D), lambda i: (0, 0)),
            pl.BlockSpec((1, D), lambda i: (0, 0)),
        ],
        out_specs=pl.BlockSpec((bn, D), lambda i: (i, 0)),
        out_shape=jax.ShapeDtypeStruct((N, D), jnp.float32),
    )(partials, wo, bo.reshape(1, D), wp, bp.reshape(1, D))


def _sc_aggregate(hv, he, src, dst, zeros):
    mesh = plsc.VectorSubcoreMesh(core_axis_name="c", subcore_axis_name="s")

    @functools.partial(
        pl.kernel,
        mesh=mesh,
        out_type=jax.ShapeDtypeStruct((2 * _NPAD, H), jnp.float32),
        scratch_types=[
            pltpu.VMEM((_C,), jnp.int32),
            pltpu.VMEM((_C,), jnp.int32),
            pltpu.VMEM((_C, H), jnp.float32),
            pltpu.VMEM((_C, H), jnp.float32),
            pltpu.VMEM_SHARED((_NPAD, H), jnp.float32),
            pltpu.SemaphoreType.DMA,
        ],
        compiler_params=pltpu.CompilerParams(use_tc_tiling_on_sc=False),
    )
    def body(hv_hbm, he_hbm, src_hbm, dst_hbm, z_hbm, out_hbm,
             src_v, dst_v, rows_v, he_v, acc, sem):
        cid = lax.axis_index("c")
        sid = lax.axis_index("s")
        g = cid * 16 + sid

        pltpu.sync_copy(
            z_hbm.at[pl.ds(sid * _ROWS_PT, _ROWS_PT)],
            acc.at[pl.ds(sid * _ROWS_PT, _ROWS_PT)],
        )
        plsc.subcore_barrier()

        def chunk(i, carry):
            ck = g + _NTILES * i

            @pl.when(ck < _NCHUNK)
            def _():
                base = ck * _C
                pltpu.sync_copy(src_hbm.at[pl.ds(base, _C)], src_v)
                pltpu.sync_copy(dst_hbm.at[pl.ds(base, _C)], dst_v)
                pltpu.async_copy(hv_hbm.at[src_v], rows_v, sem).wait()
                pltpu.sync_copy(he_hbm.at[pl.ds(base, _C)], he_v)

                def row(r, c2):
                    for j in range(H // 16):
                        sl = pl.ds(j * 16, 16)
                        rows_v[r, sl] = rows_v[r, sl] * he_v[r, sl]
                    return c2

                lax.fori_loop(0, _C, row, 0)
                pltpu.sync_copy(rows_v, acc.at[dst_v], add=True)

            return carry

        lax.fori_loop(0, (_NCHUNK + _NTILES - 1) // _NTILES, chunk, 0)
        plsc.subcore_barrier()

        pltpu.sync_copy(
            acc.at[pl.ds(sid * _ROWS_PT, _ROWS_PT)],
            out_hbm.at[pl.ds(cid * _NPAD + sid * _ROWS_PT, _ROWS_PT)],
        )

    return body(hv, he, src, dst, zeros)


def kernel(node_feats, edge_feats, edge_index, Wn, bn, We1, be1, We2, be2,
           Wo, bo, Wp, bp):
    src = edge_index[0]
    dst = edge_index[1]
    hv = _node_proj(node_feats, Wn, bn)
    he = _edge_mlp(edge_feats, We1, be1, We2, be2)
    zeros = jnp.zeros((_NPAD, H), jnp.float32)
    partials = _sc_aggregate(hv, he, src, dst, zeros).reshape(2, _NPAD, H)
    return _final(partials, Wo, bo, Wp, bp)


# speedup vs baseline: 3.2003x; 1.3645x over previous
import functools

import jax
import jax.numpy as jnp
from jax import lax
from jax.experimental import pallas as pl
from jax.experimental.pallas import tpu as pltpu
from jax.experimental.pallas import tpu_sc as plsc

N = 10000
E = 320000
D = 128
DE = 16
H = 64
MAX_NEIGHBORS = 32.0

_NTILES = 32
_C = 128
_NCHUNK = E // _C
_ROWS_PT = 632
_NPAD = 16 * _ROWS_PT


def _ssp(x):
    return jax.nn.softplus(x) - jnp.log(2.0)


def _node_proj_body(x_ref, w_ref, b_ref, o_ref):
    o_ref[...] = (
        jnp.dot(x_ref[...], w_ref[...], preferred_element_type=jnp.float32)
        + b_ref[...]
    )


def _node_proj(x, w, b):
    bn = 2000
    return pl.pallas_call(
        _node_proj_body,
        grid=(N // bn,),
        in_specs=[
            pl.BlockSpec((bn, D), lambda i: (i, 0)),
            pl.BlockSpec((D, H), lambda i: (0, 0)),
            pl.BlockSpec((1, H), lambda i: (0, 0)),
        ],
        out_specs=pl.BlockSpec((bn, H), lambda i: (i, 0)),
        out_shape=jax.ShapeDtypeStruct((N, H), jnp.float32),
    )(x, w, b.reshape(1, H))


def _edge_mlp_body(ef_ref, w1_ref, b1_ref, w2_ref, b2_ref, o_ref):
    h = (
        jnp.dot(ef_ref[...], w1_ref[...], preferred_element_type=jnp.float32)
        + b1_ref[...]
    )
    h = _ssp(h)
    h = jnp.dot(h, w2_ref[...], preferred_element_type=jnp.float32) + b2_ref[...]
    o_ref[...] = _ssp(h)


def _edge_mlp(ef, w1, b1, w2, b2):
    be = 8000
    return pl.pallas_call(
        _edge_mlp_body,
        grid=(E // be,),
        in_specs=[
            pl.BlockSpec((be, DE), lambda i: (i, 0)),
            pl.BlockSpec((DE, H), lambda i: (0, 0)),
            pl.BlockSpec((1, H), lambda i: (0, 0)),
            pl.BlockSpec((H, H), lambda i: (0, 0)),
            pl.BlockSpec((1, H), lambda i: (0, 0)),
        ],
        out_specs=pl.BlockSpec((be, H), lambda i: (i, 0)),
        out_shape=jax.ShapeDtypeStruct((E, H), jnp.float32),
    )(ef, w1, b1.reshape(1, H), w2, b2.reshape(1, H))


def _final_body(p_ref, wo_ref, bo_ref, wp_ref, bp_ref, o_ref):
    h = p_ref[0] + p_ref[1]
    h = _ssp(
        jnp.dot(h, wo_ref[...], preferred_element_type=jnp.float32) + bo_ref[...]
    ) * (1.0 / MAX_NEIGHBORS)
    o_ref[...] = (
        jnp.dot(h, wp_ref[...], preferred_element_type=jnp.float32) + bp_ref[...]
    )


def _final(partials, wo, bo, wp, bp):
    bn = 2000
    return pl.pallas_call(
        _final_body,
        grid=(N // bn,),
        in_specs=[
            pl.BlockSpec((2, bn, H), lambda i: (0, i, 0)),
            pl.BlockSpec((H, D), lambda i: (0, 0)),
            pl.BlockSpec((1, D), lambda i: (0, 0)),
            pl.BlockSpec((D, D), lambda i: (0, 0)),
            pl.BlockSpec((1, D), lambda i: (0, 0)),
        ],
        out_specs=pl.BlockSpec((bn, D), lambda i: (i, 0)),
        out_shape=jax.ShapeDtypeStruct((N, D), jnp.float32),
    )(partials, wo, bo.reshape(1, D), wp, bp.reshape(1, D))


_SLOTS = 80
_IDXROWS = 2504


def _sc_aggregate(hv, he, src2d, dst2d, zeros):
    mesh = plsc.VectorSubcoreMesh(core_axis_name="c", subcore_axis_name="s")

    @functools.partial(
        pl.kernel,
        mesh=mesh,
        out_type=jax.ShapeDtypeStruct((2 * _NPAD, H), jnp.float32),
        scratch_types=[
            pltpu.VMEM((_SLOTS, _C), jnp.int32),
            pltpu.VMEM((_SLOTS, _C), jnp.int32),
            pltpu.VMEM((_C, H), jnp.float32),
            pltpu.VMEM((_C, H), jnp.float32),
            pltpu.VMEM((_C, H), jnp.float32),
            pltpu.VMEM((_C, H), jnp.float32),
            pltpu.VMEM_SHARED((_NPAD, H), jnp.float32),
            pltpu.SemaphoreType.DMA,
            pltpu.SemaphoreType.DMA,
            pltpu.SemaphoreType.DMA,
            pltpu.SemaphoreType.DMA,
        ],
        compiler_params=pltpu.CompilerParams(use_tc_tiling_on_sc=False),
    )
    def body(hv_hbm, he_hbm, src_hbm, dst_hbm, z_hbm, out_hbm,
             src_all, dst_all, rows0, rows1, he0, he1, acc,
             sg0, sg1, sh0, sh1):
        cid = lax.axis_index("c")
        sid = lax.axis_index("s")
        g = cid * 16 + sid
        r0 = g * 78 + jnp.minimum(g, 4)
        nreal = 78 + jnp.where(g < 4, 1, 0)

        rows_b = (rows0, rows1)
        he_b = (he0, he1)
        sg_b = (sg0, sg1)
        sh_b = (sh0, sh1)

        pltpu.sync_copy(src_hbm.at[pl.ds(r0, _SLOTS)], src_all)
        pltpu.sync_copy(dst_hbm.at[pl.ds(r0, _SLOTS)], dst_all)

        def issue(i, b):
            row_cl = jnp.minimum(r0 + i, _NCHUNK - 1)
            pltpu.async_copy(he_hbm.at[pl.ds(row_cl * _C, _C)], he_b[b], sh_b[b])
            pltpu.async_copy(hv_hbm.at[src_all.at[i]], rows_b[b], sg_b[b])

        issue(0, 0)
        issue(1, 1)

        pltpu.sync_copy(
            z_hbm.at[pl.ds(sid * _ROWS_PT, _ROWS_PT)],
            acc.at[pl.ds(sid * _ROWS_PT, _ROWS_PT)],
        )
        plsc.subcore_barrier()

        def step(io, carry):
            for b in range(2):
                i = io * 2 + b
                pltpu.make_async_copy(
                    he_hbm.at[pl.ds(0, _C)], he_b[b], sh_b[b]).wait()
                pltpu.make_async_copy(
                    hv_hbm.at[src_all.at[0]], rows_b[b], sg_b[b]).wait()

                rv, hv_ = rows_b[b], he_b[b]

                def mul2(q, c2, rv=rv, hv_=hv_):
                    for rr in range(2):
                        r = q * 2 + rr
                        for j in range(H // 16):
                            sl = pl.ds(j * 16, 16)
                            rv[r, sl] = rv[r, sl] * hv_[r, sl]
                    return c2

                lax.fori_loop(0, _C // 2, mul2, 0)

                @pl.when(i < nreal)
                def _(b=b, i=i):
                    pltpu.sync_copy(rows_b[b], acc.at[dst_all.at[i]], add=True)

                @pl.when(i + 2 < _SLOTS)
                def _(b=b, i=i):
                    issue(i + 2, b)
            return carry

        lax.fori_loop(0, _SLOTS // 2, step, 0)
        plsc.subcore_barrier()

        pltpu.sync_copy(
            acc.at[pl.ds(sid * _ROWS_PT, _ROWS_PT)],
            out_hbm.at[pl.ds(cid * _NPAD + sid * _ROWS_PT, _ROWS_PT)],
        )

    return body(hv, he, src2d, dst2d, zeros)


def kernel(node_feats, edge_feats, edge_index, Wn, bn, We1, be1, We2, be2,
           Wo, bo, Wp, bp):
    ei_pad = jnp.pad(edge_index, ((0, 0), (0, _IDXROWS * _C - E)))
    src2d = ei_pad[0].reshape(_IDXROWS, _C)
    dst2d = ei_pad[1].reshape(_IDXROWS, _C)
    hv = _node_proj(node_feats, Wn, bn)
    he = _edge_mlp(edge_feats, We1, be1, We2, be2)
    zeros = jnp.zeros((_NPAD, H), jnp.float32)
    partials = _sc_aggregate(hv, he, src2d, dst2d, zeros).reshape(2, _NPAD, H)
    return _final(partials, Wo, bo, Wp, bp)


# speedup vs baseline: 5.1088x; 1.5964x over previous
import functools

import jax
import jax.numpy as jnp
from jax import lax
from jax.experimental import pallas as pl
from jax.experimental.pallas import tpu as pltpu
from jax.experimental.pallas import tpu_sc as plsc

N = 10000
E = 320000
D = 128
DE = 16
H = 64
MAX_NEIGHBORS = 32.0

_NTILES = 32
_C = 128
_NCHUNK = E // _C
_ROWS_PT = 632
_NPAD = 16 * _ROWS_PT


def _ssp(x):
    return jax.nn.softplus(x) - jnp.log(2.0)


_LOG2E = 1.4426950408889634
_LN2 = 0.6931471805599453


def _ssp_fast(x):
    a = x * _LOG2E
    t = jnp.exp2(-jnp.abs(a))
    return (jnp.maximum(a, 0.0) + jnp.log2(1.0 + t) - 1.0) * _LN2


def _node_proj_body(x_ref, w_ref, b_ref, o_ref):
    o_ref[...] = (
        jnp.dot(x_ref[...], w_ref[...], preferred_element_type=jnp.float32)
        + b_ref[...]
    )


def _node_proj(x, w, b):
    bn = 2000
    return pl.pallas_call(
        _node_proj_body,
        grid=(N // bn,),
        in_specs=[
            pl.BlockSpec((bn, D), lambda i: (i, 0)),
            pl.BlockSpec((D, H), lambda i: (0, 0)),
            pl.BlockSpec((1, H), lambda i: (0, 0)),
        ],
        out_specs=pl.BlockSpec((bn, H), lambda i: (i, 0)),
        out_shape=jax.ShapeDtypeStruct((N, H), jnp.float32),
    )(x, w, b.reshape(1, H))


_E8 = E // 8
_E2 = E // 2


def _edge_mlp_body(ef_ref, w1_ref, b1_ref, w2_ref, b2_ref, o_ref):
    z1 = (
        jnp.dot(ef_ref[...], w1_ref[...], preferred_element_type=jnp.float32)
        + b1_ref[...]
    )
    h1 = _ssp_fast(z1)
    h1p2 = h1.reshape(h1.shape[0] * 4, 128)
    z2 = (
        jnp.dot(h1p2, w2_ref[...], preferred_element_type=jnp.float32)
        + b2_ref[...]
    )
    o_ref[...] = _ssp_fast(z2)


def _edge_mlp(ef, w1, b1, w2, b2):
    be8 = 1000
    w1big = jnp.kron(jnp.eye(8, dtype=jnp.float32), w1)
    b1big = jnp.tile(b1, 8).reshape(1, 8 * H)
    w2big = jnp.kron(jnp.eye(2, dtype=jnp.float32), w2)
    b2big = jnp.tile(b2, 2).reshape(1, 2 * H)
    return pl.pallas_call(
        _edge_mlp_body,
        grid=(_E8 // be8,),
        in_specs=[
            pl.BlockSpec((be8, 128), lambda i: (i, 0)),
            pl.BlockSpec((128, 8 * H), lambda i: (0, 0)),
            pl.BlockSpec((1, 8 * H), lambda i: (0, 0)),
            pl.BlockSpec((128, 2 * H), lambda i: (0, 0)),
            pl.BlockSpec((1, 2 * H), lambda i: (0, 0)),
        ],
        out_specs=pl.BlockSpec((4 * be8, 128), lambda i: (i, 0)),
        out_shape=jax.ShapeDtypeStruct((_E2, 128), jnp.float32),
    )(ef.reshape(_E8, 128), w1big, b1big, w2big, b2big)


def _final_body(p_ref, wo_ref, bo_ref, wp_ref, bp_ref, o_ref):
    h = p_ref[0] + p_ref[1]
    h = _ssp_fast(
        jnp.dot(h, wo_ref[...], preferred_element_type=jnp.float32) + bo_ref[...]
    ) * (1.0 / MAX_NEIGHBORS)
    o_ref[...] = (
        jnp.dot(h, wp_ref[...], preferred_element_type=jnp.float32) + bp_ref[...]
    )


def _final(partials, wo, bo, wp, bp):
    bn = 2000
    return pl.pallas_call(
        _final_body,
        grid=(N // bn,),
        in_specs=[
            pl.BlockSpec((2, bn, H), lambda i: (0, i, 0)),
            pl.BlockSpec((H, D), lambda i: (0, 0)),
            pl.BlockSpec((1, D), lambda i: (0, 0)),
            pl.BlockSpec((D, D), lambda i: (0, 0)),
            pl.BlockSpec((1, D), lambda i: (0, 0)),
        ],
        out_specs=pl.BlockSpec((bn, D), lambda i: (i, 0)),
        out_shape=jax.ShapeDtypeStruct((N, D), jnp.float32),
    )(partials, wo, bo.reshape(1, D), wp, bp.reshape(1, D))


_SLOTS = 80
_IDXROWS = 2504


def _sc_aggregate(hv, he, src2d, dst2d, zeros):
    mesh = plsc.VectorSubcoreMesh(core_axis_name="c", subcore_axis_name="s")

    @functools.partial(
        pl.kernel,
        mesh=mesh,
        out_type=jax.ShapeDtypeStruct((2 * _NPAD, H), jnp.float32),
        scratch_types=[
            pltpu.VMEM((_SLOTS, _C), jnp.int32),
            pltpu.VMEM((_SLOTS, _C), jnp.int32),
            pltpu.VMEM((_C, H), jnp.float32),
            pltpu.VMEM((_C, H), jnp.float32),
            pltpu.VMEM((_C // 2, 128), jnp.float32),
            pltpu.VMEM((_C // 2, 128), jnp.float32),
            pltpu.VMEM_SHARED((_NPAD, H), jnp.float32),
            pltpu.SemaphoreType.DMA,
            pltpu.SemaphoreType.DMA,
            pltpu.SemaphoreType.DMA,
            pltpu.SemaphoreType.DMA,
        ],
        compiler_params=pltpu.CompilerParams(use_tc_tiling_on_sc=False),
    )
    def body(hv_hbm, he_hbm, src_hbm, dst_hbm, z_hbm, out_hbm,
             src_all, dst_all, rows0, rows1, he0, he1, acc,
             sg0, sg1, sh0, sh1):
        cid = lax.axis_index("c")
        sid = lax.axis_index("s")
        g = cid * 16 + sid
        r0 = g * 78 + jnp.minimum(g, 4)
        nreal = 78 + jnp.where(g < 4, 1, 0)

        rows_b = (rows0, rows1)
        he_b = (he0, he1)
        sg_b = (sg0, sg1)
        sh_b = (sh0, sh1)

        pltpu.sync_copy(src_hbm.at[pl.ds(r0, _SLOTS)], src_all)
        pltpu.sync_copy(dst_hbm.at[pl.ds(r0, _SLOTS)], dst_all)

        def issue(i, b):
            row_cl = jnp.minimum(r0 + i, _NCHUNK - 1)
            pltpu.async_copy(
                he_hbm.at[pl.ds(row_cl * (_C // 2), _C // 2)], he_b[b], sh_b[b])
            pltpu.async_copy(hv_hbm.at[src_all.at[i]], rows_b[b], sg_b[b])

        issue(0, 0)
        issue(1, 1)

        pltpu.sync_copy(
            z_hbm.at[pl.ds(sid * _ROWS_PT, _ROWS_PT)],
            acc.at[pl.ds(sid * _ROWS_PT, _ROWS_PT)],
        )
        plsc.subcore_barrier()

        def step(io, carry):
            for b in range(2):
                i = io * 2 + b
                pltpu.make_async_copy(
                    he_hbm.at[pl.ds(0, _C // 2)], he_b[b], sh_b[b]).wait()
                pltpu.make_async_copy(
                    hv_hbm.at[src_all.at[0]], rows_b[b], sg_b[b]).wait()

                rv, hv_ = rows_b[b], he_b[b]

                def mul2(q, c2, rv=rv, hv_=hv_):
                    for rr in range(2):
                        r = q * 2 + rr
                        for j in range(H // 16):
                            sl = pl.ds(j * 16, 16)
                            sl2 = pl.ds(rr * H + j * 16, 16)
                            rv[r, sl] = rv[r, sl] * hv_[q, sl2]
                    return c2

                lax.fori_loop(0, _C // 2, mul2, 0)

                @pl.when(i < nreal)
                def _(b=b, i=i):
                    pltpu.sync_copy(rows_b[b], acc.at[dst_all.at[i]], add=True)

                @pl.when(i + 2 < _SLOTS)
                def _(b=b, i=i):
                    issue(i + 2, b)
            return carry

        lax.fori_loop(0, _SLOTS // 2, step, 0)
        plsc.subcore_barrier()

        pltpu.sync_copy(
            acc.at[pl.ds(sid * _ROWS_PT, _ROWS_PT)],
            out_hbm.at[pl.ds(cid * _NPAD + sid * _ROWS_PT, _ROWS_PT)],
        )

    return body(hv, he, src2d, dst2d, zeros)


def kernel(node_feats, edge_feats, edge_index, Wn, bn, We1, be1, We2, be2,
           Wo, bo, Wp, bp):
    ei_pad = jnp.pad(edge_index, ((0, 0), (0, _IDXROWS * _C - E)))
    src2d = ei_pad[0].reshape(_IDXROWS, _C)
    dst2d = ei_pad[1].reshape(_IDXROWS, _C)
    hv = _node_proj(node_feats, Wn, bn)
    he = _edge_mlp(edge_feats, We1, be1, We2, be2)
    zeros = jnp.zeros((_NPAD, H), jnp.float32)
    partials = _sc_aggregate(hv, he, src2d, dst2d, zeros).reshape(2, _NPAD, H)
    return _final(partials, Wo, bo, Wp, bp)


# speedup vs baseline: 5.7695x; 1.1293x over previous
import functools

import jax
import jax.numpy as jnp
from jax import lax
from jax.experimental import pallas as pl
from jax.experimental.pallas import tpu as pltpu
from jax.experimental.pallas import tpu_sc as plsc

N = 10000
E = 320000
D = 128
DE = 16
H = 64
MAX_NEIGHBORS = 32.0

_NTILES = 32
_C = 128
_NCHUNK = E // _C
_ROWS_PT = 632
_NPAD = 16 * _ROWS_PT


def _ssp(x):
    return jax.nn.softplus(x) - jnp.log(2.0)


_LOG2E = 1.4426950408889634
_LN2 = 0.6931471805599453


def _ssp_fast(x):
    a = x * _LOG2E
    t = jnp.exp2(-jnp.abs(a))
    return (jnp.maximum(a, 0.0) + jnp.log2(1.0 + t) - 1.0) * _LN2


def _node_proj_body(x_ref, w_ref, b_ref, o_ref):
    o_ref[...] = (
        jnp.dot(x_ref[...], w_ref[...], preferred_element_type=jnp.float32)
        + b_ref[...]
    )


def _node_proj(x, w, b):
    bn2 = 1000
    wbd = jnp.kron(jnp.eye(2, dtype=jnp.float32), w)
    bbd = jnp.tile(b, 2).reshape(1, 2 * H)
    return pl.pallas_call(
        _node_proj_body,
        grid=(N // 2 // bn2,),
        in_specs=[
            pl.BlockSpec((bn2, 2 * D), lambda i: (i, 0)),
            pl.BlockSpec((2 * D, 2 * H), lambda i: (0, 0)),
            pl.BlockSpec((1, 2 * H), lambda i: (0, 0)),
        ],
        out_specs=pl.BlockSpec((bn2, 2 * H), lambda i: (i, 0)),
        out_shape=jax.ShapeDtypeStruct((N // 2, 2 * H), jnp.float32),
    )(x.reshape(N // 2, 2 * D), wbd, bbd)


_E2 = E // 2


def _edge_mlp_half(eft, w1, b1, w2, b2):
    z1 = lax.dot_general(
        eft, w1, (((0,), (0,)), ((), ())),
        preferred_element_type=jnp.float32,
    ) + b1
    h1 = _ssp_fast(z1)
    z2 = jnp.dot(h1, w2, preferred_element_type=jnp.float32) + b2
    return _ssp_fast(z2)


def _edge_mlp_body(efta_ref, eftb_ref, w1_ref, b1_ref, w2_ref, b2_ref, o_ref):
    w1, b1 = w1_ref[...], b1_ref[...]
    w2, b2 = w2_ref[...], b2_ref[...]
    hea = _edge_mlp_half(efta_ref[...], w1, b1, w2, b2)
    heb = _edge_mlp_half(eftb_ref[...], w1, b1, w2, b2)
    o_ref[...] = jnp.concatenate([hea, heb], axis=1)


def _edge_mlp(ef, w1, b1, w2, b2):
    be = 6400
    nblk = _E2 // be
    eft = ef.T
    return pl.pallas_call(
        _edge_mlp_body,
        grid=(nblk,),
        in_specs=[
            pl.BlockSpec((DE, be), lambda i: (0, i)),
            pl.BlockSpec((DE, be), lambda i, nblk=nblk: (0, i + nblk)),
            pl.BlockSpec((DE, H), lambda i: (0, 0)),
            pl.BlockSpec((1, H), lambda i: (0, 0)),
            pl.BlockSpec((H, H), lambda i: (0, 0)),
            pl.BlockSpec((1, H), lambda i: (0, 0)),
        ],
        out_specs=pl.BlockSpec((be, 128), lambda i: (i, 0)),
        out_shape=jax.ShapeDtypeStruct((_E2, 128), jnp.float32),
    )(eft, eft, w1, b1.reshape(1, H), w2, b2.reshape(1, H))


def _final_body(p_ref, wo_ref, bo_ref, wp_ref, bp_ref, o_ref):
    h = p_ref[0] + p_ref[1]
    h = _ssp_fast(
        jnp.dot(h, wo_ref[...], preferred_element_type=jnp.float32) + bo_ref[...]
    ) * (1.0 / MAX_NEIGHBORS)
    o_ref[...] = (
        jnp.dot(h, wp_ref[...], preferred_element_type=jnp.float32)
        + bp_ref[...]
    )


def _final(partials, wo, bo, wp, bp):
    bn = 2000
    return pl.pallas_call(
        _final_body,
        grid=(N // bn,),
        in_specs=[
            pl.BlockSpec((2, bn, H), lambda i: (0, i, 0)),
            pl.BlockSpec((H, D), lambda i: (0, 0)),
            pl.BlockSpec((1, D), lambda i: (0, 0)),
            pl.BlockSpec((D, D), lambda i: (0, 0)),
            pl.BlockSpec((1, D), lambda i: (0, 0)),
        ],
        out_specs=pl.BlockSpec((bn, D), lambda i: (i, 0)),
        out_shape=jax.ShapeDtypeStruct((N, D), jnp.float32),
    )(partials, wo, bo.reshape(1, D), wp, bp.reshape(1, D))


_SLOTS = 80


def _sc_aggregate(hv, he, src2d, dst2d, zeros):
    mesh = plsc.VectorSubcoreMesh(core_axis_name="c", subcore_axis_name="s")

    @functools.partial(
        pl.kernel,
        mesh=mesh,
        out_type=jax.ShapeDtypeStruct((2 * _NPAD, H), jnp.float32),
        scratch_types=[
            pltpu.VMEM((_SLOTS, _C), jnp.int32),
            pltpu.VMEM((_SLOTS, _C), jnp.int32),
            pltpu.VMEM((_C, H), jnp.float32),
            pltpu.VMEM((_C, H), jnp.float32),
            pltpu.VMEM((_C, H), jnp.float32),
            pltpu.VMEM((_C, H), jnp.float32),
            pltpu.VMEM_SHARED((_NPAD, H), jnp.float32),
            pltpu.SemaphoreType.DMA,
            pltpu.SemaphoreType.DMA,
            pltpu.SemaphoreType.DMA,
            pltpu.SemaphoreType.DMA,
        ],
        compiler_params=pltpu.CompilerParams(use_tc_tiling_on_sc=False),
    )
    def body(hv_hbm, he_hbm, src_hbm, dst_hbm, z_hbm, out_hbm,
             src_all, dst_all, rows0, rows1, he0, he1, acc,
             sg0, sg1, sh0, sh1):
        cid = lax.axis_index("c")
        sid = lax.axis_index("s")
        g = cid * 16 + sid
        r0 = g * 78 + jnp.minimum(g, 4)
        nreal = 78 + jnp.where(g < 4, 1, 0)
        wstart = jnp.minimum(r0, _NCHUNK - _SLOTS)
        off = r0 - wstart

        rows_b = (rows0, rows1)
        he_b = (he0, he1)
        sg_b = (sg0, sg1)
        sh_b = (sh0, sh1)

        pltpu.sync_copy(src_hbm.at[pl.ds(wstart, _SLOTS)], src_all)
        pltpu.sync_copy(dst_hbm.at[pl.ds(wstart, _SLOTS)], dst_all)

        def idxrow(i):
            return jnp.minimum(off + i, _SLOTS - 1)

        def issue(i, b):
            row_cl = jnp.minimum(r0 + i, _NCHUNK - 1)
            rsel = jnp.where(row_cl < _NCHUNK // 2,
                             row_cl, row_cl - _NCHUNK // 2) * _C
            csel = jnp.where(row_cl < _NCHUNK // 2, 0, H)
            pltpu.async_copy(
                he_hbm.at[pl.ds(rsel, _C), pl.ds(csel, H)], he_b[b], sh_b[b])
            pltpu.async_copy(hv_hbm.at[src_all.at[idxrow(i)]], rows_b[b], sg_b[b])

        issue(0, 0)
        issue(1, 1)

        pltpu.sync_copy(
            z_hbm.at[pl.ds(sid * _ROWS_PT, _ROWS_PT)],
            acc.at[pl.ds(sid * _ROWS_PT, _ROWS_PT)],
        )
        plsc.subcore_barrier()

        def step(io, carry):
            for b in range(2):
                i = io * 2 + b
                pltpu.make_async_copy(
                    he_hbm.at[pl.ds(0, _C), pl.ds(0, H)], he_b[b], sh_b[b]).wait()
                pltpu.make_async_copy(
                    hv_hbm.at[src_all.at[0]], rows_b[b], sg_b[b]).wait()

                rv, hv_ = rows_b[b], he_b[b]

                def mul2(q, c2, rv=rv, hv_=hv_):
                    for rr in range(2):
                        r = q * 2 + rr
                        for j in range(H // 16):
                            sl = pl.ds(j * 16, 16)
                            rv[r, sl] = rv[r, sl] * hv_[r, sl]
                    return c2

                lax.fori_loop(0, _C // 2, mul2, 0)

                @pl.when(i < nreal)
                def _(b=b, i=i):
                    pltpu.sync_copy(rows_b[b], acc.at[dst_all.at[idxrow(i)]],
                                    add=True)

                @pl.when(i + 2 < _SLOTS)
                def _(b=b, i=i):
                    issue(i + 2, b)
            return carry

        lax.fori_loop(0, _SLOTS // 2, step, 0)
        plsc.subcore_barrier()

        pltpu.sync_copy(
            acc.at[pl.ds(sid * _ROWS_PT, _ROWS_PT)],
            out_hbm.at[pl.ds(cid * _NPAD + sid * _ROWS_PT, _ROWS_PT)],
        )

    return body(hv, he, src2d, dst2d, zeros)


def kernel(node_feats, edge_feats, edge_index, Wn, bn, We1, be1, We2, be2,
           Wo, bo, Wp, bp):
    src2d = edge_index[0].reshape(_NCHUNK, _C)
    dst2d = edge_index[1].reshape(_NCHUNK, _C)
    hv = _node_proj(node_feats, Wn, bn).reshape(N, H)
    he = _edge_mlp(edge_feats, We1, be1, We2, be2)
    zeros = jnp.zeros((_NPAD, H), jnp.float32)
    partials = _sc_aggregate(hv, he, src2d, dst2d, zeros)
    return _final(partials.reshape(2, _NPAD, H), Wo, bo, Wp, bp)


# speedup vs baseline: 5.8138x; 1.0077x over previous
import functools

import jax
import jax.numpy as jnp
from jax import lax
from jax.experimental import pallas as pl
from jax.experimental.pallas import tpu as pltpu
from jax.experimental.pallas import tpu_sc as plsc

N = 10000
E = 320000
D = 128
DE = 16
H = 64
MAX_NEIGHBORS = 32.0

_NTILES = 32
_C = 128
_NCHUNK = E // _C
_ROWS_PT = 632
_NPAD = 16 * _ROWS_PT


def _ssp(x):
    return jax.nn.softplus(x) - jnp.log(2.0)


_LOG2E = 1.4426950408889634
_LN2 = 0.6931471805599453


def _ssp_fast(x):
    a = x * _LOG2E
    t = jnp.exp2(-jnp.abs(a))
    return (jnp.maximum(a, 0.0) + jnp.log2(1.0 + t) - 1.0) * _LN2


def _node_proj_body(x_ref, w_ref, b_ref, o_ref):
    o_ref[...] = (
        jnp.dot(x_ref[...], w_ref[...], preferred_element_type=jnp.float32)
        + b_ref[...]
    )


def _node_proj(x, w, b):
    bn2 = 1000
    wbd = jnp.kron(jnp.eye(2, dtype=jnp.float32), w)
    bbd = jnp.tile(b, 2).reshape(1, 2 * H)
    return pl.pallas_call(
        _node_proj_body,
        grid=(N // 2 // bn2,),
        in_specs=[
            pl.BlockSpec((bn2, 2 * D), lambda i: (i, 0)),
            pl.BlockSpec((2 * D, 2 * H), lambda i: (0, 0)),
            pl.BlockSpec((1, 2 * H), lambda i: (0, 0)),
        ],
        out_specs=pl.BlockSpec((bn2, 2 * H), lambda i: (i, 0)),
        out_shape=jax.ShapeDtypeStruct((N // 2, 2 * H), jnp.float32),
    )(x.reshape(N // 2, 2 * D), wbd, bbd)


_E2 = E // 2


def _edge_mlp_half(eft, w1, b1, w2, b2):
    z1 = lax.dot_general(
        eft, w1, (((0,), (0,)), ((), ())),
        preferred_element_type=jnp.float32,
    ) + b1
    h1 = _ssp_fast(z1)
    z2 = jnp.dot(h1, w2, preferred_element_type=jnp.float32) + b2
    return _ssp_fast(z2)


def _edge_mlp_body(efta_ref, eftb_ref, w1_ref, b1_ref, w2_ref, b2_ref, o_ref):
    w1, b1 = w1_ref[...], b1_ref[...]
    w2, b2 = w2_ref[...], b2_ref[...]
    hea = _edge_mlp_half(efta_ref[...], w1, b1, w2, b2)
    heb = _edge_mlp_half(eftb_ref[...], w1, b1, w2, b2)
    o_ref[...] = jnp.concatenate([hea, heb], axis=1)


def _edge_mlp(ef, w1, b1, w2, b2):
    be = 16000
    nblk = _E2 // be
    eft = ef.T
    return pl.pallas_call(
        _edge_mlp_body,
        grid=(nblk,),
        in_specs=[
            pl.BlockSpec((DE, be), lambda i: (0, i)),
            pl.BlockSpec((DE, be), lambda i, nblk=nblk: (0, i + nblk)),
            pl.BlockSpec((DE, H), lambda i: (0, 0)),
            pl.BlockSpec((1, H), lambda i: (0, 0)),
            pl.BlockSpec((H, H), lambda i: (0, 0)),
            pl.BlockSpec((1, H), lambda i: (0, 0)),
        ],
        out_specs=pl.BlockSpec((be, 128), lambda i: (i, 0)),
        out_shape=jax.ShapeDtypeStruct((_E2, 128), jnp.float32),
        compiler_params=pltpu.CompilerParams(fuse_transposed_lhs_in_matmul=True),
    )(eft, eft, w1, b1.reshape(1, H), w2, b2.reshape(1, H))


def _final_body(p_ref, wo_ref, bo_ref, wp_ref, bp_ref, o_ref):
    h = p_ref[0] + p_ref[1]
    h = _ssp_fast(
        jnp.dot(h, wo_ref[...], preferred_element_type=jnp.float32) + bo_ref[...]
    ) * (1.0 / MAX_NEIGHBORS)
    o_ref[...] = (
        jnp.dot(h, wp_ref[...], preferred_element_type=jnp.float32)
        + bp_ref[...]
    )


def _final(partials, wo, bo, wp, bp):
    bn = 2000
    return pl.pallas_call(
        _final_body,
        grid=(N // bn,),
        in_specs=[
            pl.BlockSpec((2, bn, H), lambda i: (0, i, 0)),
            pl.BlockSpec((H, D), lambda i: (0, 0)),
            pl.BlockSpec((1, D), lambda i: (0, 0)),
            pl.BlockSpec((D, D), lambda i: (0, 0)),
            pl.BlockSpec((1, D), lambda i: (0, 0)),
        ],
        out_specs=pl.BlockSpec((bn, D), lambda i: (i, 0)),
        out_shape=jax.ShapeDtypeStruct((N, D), jnp.float32),
    )(partials, wo, bo.reshape(1, D), wp, bp.reshape(1, D))


_SLOTS = 80


def _sc_aggregate(hv, he, src2d, dst2d, zeros):
    mesh = plsc.VectorSubcoreMesh(core_axis_name="c", subcore_axis_name="s")

    @functools.partial(
        pl.kernel,
        mesh=mesh,
        out_type=jax.ShapeDtypeStruct((2 * _NPAD, H), jnp.float32),
        scratch_types=[
            pltpu.VMEM((_SLOTS, _C), jnp.int32),
            pltpu.VMEM((_SLOTS, _C), jnp.int32),
            pltpu.VMEM((_C, H), jnp.float32),
            pltpu.VMEM((_C, H), jnp.float32),
            pltpu.VMEM((_C, H), jnp.float32),
            pltpu.VMEM((_C, H), jnp.float32),
            pltpu.VMEM_SHARED((_NPAD, H), jnp.float32),
            pltpu.SemaphoreType.DMA,
            pltpu.SemaphoreType.DMA,
            pltpu.SemaphoreType.DMA,
            pltpu.SemaphoreType.DMA,
        ],
        compiler_params=pltpu.CompilerParams(use_tc_tiling_on_sc=False),
    )
    def body(hv_hbm, he_hbm, src_hbm, dst_hbm, z_hbm, out_hbm,
             src_all, dst_all, rows0, rows1, he0, he1, acc,
             sg0, sg1, sh0, sh1):
        cid = lax.axis_index("c")
        sid = lax.axis_index("s")
        g = cid * 16 + sid
        r0 = g * 78 + jnp.minimum(g, 4)
        nreal = 78 + jnp.where(g < 4, 1, 0)
        wstart = jnp.minimum(r0, _NCHUNK - _SLOTS)
        off = r0 - wstart

        rows_b = (rows0, rows1)
        he_b = (he0, he1)
        sg_b = (sg0, sg1)
        sh_b = (sh0, sh1)

        pltpu.sync_copy(src_hbm.at[pl.ds(wstart, _SLOTS)], src_all)
        pltpu.sync_copy(dst_hbm.at[pl.ds(wstart, _SLOTS)], dst_all)

        def idxrow(i):
            return jnp.minimum(off + i, _SLOTS - 1)

        def issue(i, b):
            row_cl = jnp.minimum(r0 + i, _NCHUNK - 1)
            rsel = jnp.where(row_cl < _NCHUNK // 2,
                             row_cl, row_cl - _NCHUNK // 2) * _C
            csel = jnp.where(row_cl < _NCHUNK // 2, 0, H)
            pltpu.async_copy(
                he_hbm.at[pl.ds(rsel, _C), pl.ds(csel, H)], he_b[b], sh_b[b])
            pltpu.async_copy(hv_hbm.at[src_all.at[idxrow(i)]], rows_b[b], sg_b[b])

        issue(0, 0)
        issue(1, 1)

        pltpu.sync_copy(
            z_hbm.at[pl.ds(sid * _ROWS_PT, _ROWS_PT)],
            acc.at[pl.ds(sid * _ROWS_PT, _ROWS_PT)],
        )
        plsc.subcore_barrier()

        def step(io, carry):
            for b in range(2):
                i = io * 2 + b
                pltpu.make_async_copy(
                    he_hbm.at[pl.ds(0, _C), pl.ds(0, H)], he_b[b], sh_b[b]).wait()
                pltpu.make_async_copy(
                    hv_hbm.at[src_all.at[0]], rows_b[b], sg_b[b]).wait()

                rv, hv_ = rows_b[b], he_b[b]

                def mul2(q, c2, rv=rv, hv_=hv_):
                    for rr in range(2):
                        r = q * 2 + rr
                        for j in range(H // 16):
                            sl = pl.ds(j * 16, 16)
                            rv[r, sl] = rv[r, sl] * hv_[r, sl]
                    return c2

                lax.fori_loop(0, _C // 2, mul2, 0)

                @pl.when(i < nreal)
                def _(b=b, i=i):
                    pltpu.sync_copy(rows_b[b], acc.at[dst_all.at[idxrow(i)]],
                                    add=True)

                @pl.when(i + 2 < _SLOTS)
                def _(b=b, i=i):
                    issue(i + 2, b)
            return carry

        lax.fori_loop(0, _SLOTS // 2, step, 0)
        plsc.subcore_barrier()

        pltpu.sync_copy(
            acc.at[pl.ds(sid * _ROWS_PT, _ROWS_PT)],
            out_hbm.at[pl.ds(cid * _NPAD + sid * _ROWS_PT, _ROWS_PT)],
        )

    return body(hv, he, src2d, dst2d, zeros)


def kernel(node_feats, edge_feats, edge_index, Wn, bn, We1, be1, We2, be2,
           Wo, bo, Wp, bp):
    src2d = edge_index[0].reshape(_NCHUNK, _C)
    dst2d = edge_index[1].reshape(_NCHUNK, _C)
    hv = _node_proj(node_feats, Wn, bn).reshape(N, H)
    he = _edge_mlp(edge_feats, We1, be1, We2, be2)
    zeros = jnp.zeros((_NPAD, H), jnp.float32)
    partials = _sc_aggregate(hv, he, src2d, dst2d, zeros)
    return _final(partials.reshape(2, _NPAD, H), Wo, bo, Wp, bp)


# speedup vs baseline: 6.3953x; 1.1000x over previous
import functools

import jax
import jax.numpy as jnp
from jax import lax
from jax.experimental import pallas as pl
from jax.experimental.pallas import tpu as pltpu
from jax.experimental.pallas import tpu_sc as plsc

N = 10000
E = 320000
D = 128
DE = 16
H = 64
MAX_NEIGHBORS = 32.0

_NTILES = 32
_C = 128
_NCHUNK = E // _C
_ROWS_PT = 632
_NPAD = 16 * _ROWS_PT


def _ssp(x):
    return jax.nn.softplus(x) - jnp.log(2.0)


_LOG2E = 1.4426950408889634
_LN2 = 0.6931471805599453


def _ssp_fast(x):
    a = x * _LOG2E
    t = jnp.exp2(-jnp.abs(a))
    return (jnp.maximum(a, 0.0) + jnp.log2(1.0 + t) - 1.0) * _LN2


def _node_proj_body(x_ref, w_ref, b_ref, o_ref):
    o_ref[...] = (
        jnp.dot(x_ref[...], w_ref[...], preferred_element_type=jnp.float32)
        + b_ref[...]
    )


def _node_proj(x, w, b):
    bn2 = 1000
    wbd = jnp.kron(jnp.eye(2, dtype=jnp.float32), w)
    bbd = jnp.tile(b, 2).reshape(1, 2 * H)
    return pl.pallas_call(
        _node_proj_body,
        grid=(N // 2 // bn2,),
        in_specs=[
            pl.BlockSpec((bn2, 2 * D), lambda i: (i, 0)),
            pl.BlockSpec((2 * D, 2 * H), lambda i: (0, 0)),
            pl.BlockSpec((1, 2 * H), lambda i: (0, 0)),
        ],
        out_specs=pl.BlockSpec((bn2, 2 * H), lambda i: (i, 0)),
        out_shape=jax.ShapeDtypeStruct((N // 2, 2 * H), jnp.float32),
    )(x.reshape(N // 2, 2 * D), wbd, bbd)


_E2 = E // 2


def _edge_mlp_half(eft, w1, b1, w2, b2):
    z1 = lax.dot_general(
        eft, w1, (((0,), (0,)), ((), ())),
        preferred_element_type=jnp.float32,
    ) + b1
    h1 = _ssp_fast(z1)
    z2 = jnp.dot(h1, w2, preferred_element_type=jnp.float32) + b2
    return _ssp_fast(z2)


def _edge_mlp_body(efta_ref, eftb_ref, w1_ref, b1_ref, w2_ref, b2_ref, o_ref):
    w1, b1 = w1_ref[...], b1_ref[...]
    w2, b2 = w2_ref[...], b2_ref[...]
    hea = _edge_mlp_half(efta_ref[...], w1, b1, w2, b2)
    heb = _edge_mlp_half(eftb_ref[...], w1, b1, w2, b2)
    o_ref[...] = jnp.concatenate([hea, heb], axis=1)


_E4 = E // 4


def _edge_mlp(ef, w1, b1, w2, b2, part):
    be = 16000
    nblk = _E4 // be
    p0 = part * 2 * nblk
    eft = ef.T
    return pl.pallas_call(
        _edge_mlp_body,
        grid=(nblk,),
        in_specs=[
            pl.BlockSpec((DE, be), lambda i, p0=p0: (0, p0 + i)),
            pl.BlockSpec((DE, be), lambda i, p0=p0, nblk=nblk: (0, p0 + nblk + i)),
            pl.BlockSpec((DE, H), lambda i: (0, 0)),
            pl.BlockSpec((1, H), lambda i: (0, 0)),
            pl.BlockSpec((H, H), lambda i: (0, 0)),
            pl.BlockSpec((1, H), lambda i: (0, 0)),
        ],
        out_specs=pl.BlockSpec((be, 128), lambda i: (i, 0)),
        out_shape=jax.ShapeDtypeStruct((_E4, 128), jnp.float32),
        compiler_params=pltpu.CompilerParams(fuse_transposed_lhs_in_matmul=True),
    )(eft, eft, w1, b1.reshape(1, H), w2, b2.reshape(1, H))


def _final_body(pa_ref, pb_ref, wo_ref, bo_ref, wp_ref, bp_ref, o_ref):
    h = pa_ref[0] + pa_ref[1] + pb_ref[0] + pb_ref[1]
    h = _ssp_fast(
        jnp.dot(h, wo_ref[...], preferred_element_type=jnp.float32) + bo_ref[...]
    ) * (1.0 / MAX_NEIGHBORS)
    o_ref[...] = (
        jnp.dot(h, wp_ref[...], preferred_element_type=jnp.float32)
        + bp_ref[...]
    )


def _final(pa, pb, wo, bo, wp, bp):
    bn = 2000
    pspec = pl.BlockSpec((2, bn, H), lambda i: (0, i, 0))
    return pl.pallas_call(
        _final_body,
        grid=(N // bn,),
        in_specs=[
            pspec,
            pspec,
            pl.BlockSpec((H, D), lambda i: (0, 0)),
            pl.BlockSpec((1, D), lambda i: (0, 0)),
            pl.BlockSpec((D, D), lambda i: (0, 0)),
            pl.BlockSpec((1, D), lambda i: (0, 0)),
        ],
        out_specs=pl.BlockSpec((bn, D), lambda i: (i, 0)),
        out_shape=jax.ShapeDtypeStruct((N, D), jnp.float32),
    )(pa, pb, wo, bo.reshape(1, D), wp, bp.reshape(1, D))


_SLOTS = 40
_NCH = _NCHUNK // 2


def _sc_aggregate(hv, he, src2d, dst2d, zeros, part):
    mesh = plsc.VectorSubcoreMesh(core_axis_name="c", subcore_axis_name="s")

    @functools.partial(
        pl.kernel,
        mesh=mesh,
        out_type=jax.ShapeDtypeStruct((2 * _NPAD, H), jnp.float32),
        scratch_types=[
            pltpu.VMEM((_SLOTS, _C), jnp.int32),
            pltpu.VMEM((_SLOTS, _C), jnp.int32),
            pltpu.VMEM((_C, H), jnp.float32),
            pltpu.VMEM((_C, H), jnp.float32),
            pltpu.VMEM((_C, H), jnp.float32),
            pltpu.VMEM((_C, H), jnp.float32),
            pltpu.VMEM_SHARED((_NPAD, H), jnp.float32),
            pltpu.SemaphoreType.DMA,
            pltpu.SemaphoreType.DMA,
            pltpu.SemaphoreType.DMA,
            pltpu.SemaphoreType.DMA,
        ],
        compiler_params=pltpu.CompilerParams(use_tc_tiling_on_sc=False),
    )
    def body(hv_hbm, he_hbm, src_hbm, dst_hbm, z_hbm, out_hbm,
             src_all, dst_all, rows0, rows1, he0, he1, acc,
             sg0, sg1, sh0, sh1):
        cid = lax.axis_index("c")
        sid = lax.axis_index("s")
        g = cid * 16 + sid
        base = part * _NCH
        r0 = base + g * 39 + jnp.minimum(g, 2)
        nreal = 39 + jnp.where(g < 2, 1, 0)
        wstart = jnp.minimum(r0, base + _NCH - _SLOTS)
        off = r0 - wstart

        rows_b = (rows0, rows1)
        he_b = (he0, he1)
        sg_b = (sg0, sg1)
        sh_b = (sh0, sh1)

        pltpu.sync_copy(src_hbm.at[pl.ds(wstart, _SLOTS)], src_all)
        pltpu.sync_copy(dst_hbm.at[pl.ds(wstart, _SLOTS)], dst_all)

        def idxrow(i):
            return jnp.minimum(off + i, _SLOTS - 1)

        def issue(i, b):
            row_cl = jnp.minimum(r0 + i, base + _NCH - 1) - base
            rsel = jnp.where(row_cl < _NCH // 2,
                             row_cl, row_cl - _NCH // 2) * _C
            csel = jnp.where(row_cl < _NCH // 2, 0, H)
            pltpu.async_copy(
                he_hbm.at[pl.ds(rsel, _C), pl.ds(csel, H)], he_b[b], sh_b[b])
            pltpu.async_copy(hv_hbm.at[src_all.at[idxrow(i)]], rows_b[b], sg_b[b])

        issue(0, 0)
        issue(1, 1)

        pltpu.sync_copy(
            z_hbm.at[pl.ds(sid * _ROWS_PT, _ROWS_PT)],
            acc.at[pl.ds(sid * _ROWS_PT, _ROWS_PT)],
        )
        plsc.subcore_barrier()

        def step(io, carry):
            for b in range(2):
                i = io * 2 + b
                pltpu.make_async_copy(
                    he_hbm.at[pl.ds(0, _C), pl.ds(0, H)], he_b[b], sh_b[b]).wait()
                pltpu.make_async_copy(
                    hv_hbm.at[src_all.at[0]], rows_b[b], sg_b[b]).wait()

                rv, hv_ = rows_b[b], he_b[b]

                def mul2(q, c2, rv=rv, hv_=hv_):
                    for rr in range(2):
                        r = q * 2 + rr
                        for j in range(H // 16):
                            sl = pl.ds(j * 16, 16)
                            rv[r, sl] = rv[r, sl] * hv_[r, sl]
                    return c2

                lax.fori_loop(0, _C // 2, mul2, 0)

                @pl.when(i < nreal)
                def _(b=b, i=i):
                    pltpu.sync_copy(rows_b[b], acc.at[dst_all.at[idxrow(i)]],
                                    add=True)

                @pl.when(i + 2 < _SLOTS)
                def _(b=b, i=i):
                    issue(i + 2, b)
            return carry

        lax.fori_loop(0, _SLOTS // 2, step, 0)
        plsc.subcore_barrier()

        pltpu.sync_copy(
            acc.at[pl.ds(sid * _ROWS_PT, _ROWS_PT)],
            out_hbm.at[pl.ds(cid * _NPAD + sid * _ROWS_PT, _ROWS_PT)],
        )

    return body(hv, he, src2d, dst2d, zeros)


def kernel(node_feats, edge_feats, edge_index, Wn, bn, We1, be1, We2, be2,
           Wo, bo, Wp, bp):
    src2d = edge_index[0].reshape(_NCHUNK, _C)
    dst2d = edge_index[1].reshape(_NCHUNK, _C)
    hv = _node_proj(node_feats, Wn, bn).reshape(N, H)
    zeros = jnp.zeros((_NPAD, H), jnp.float32)
    he0 = _edge_mlp(edge_feats, We1, be1, We2, be2, 0)
    p0 = _sc_aggregate(hv, he0, src2d, dst2d, zeros, 0)
    he1 = _edge_mlp(edge_feats, We1, be1, We2, be2, 1)
    p1 = _sc_aggregate(hv, he1, src2d, dst2d, zeros, 1)
    return _final(p0.reshape(2, _NPAD, H), p1.reshape(2, _NPAD, H),
                  Wo, bo, Wp, bp)


# speedup vs baseline: 6.4833x; 1.0138x over previous
import functools

import jax
import jax.numpy as jnp
from jax import lax
from jax.experimental import pallas as pl
from jax.experimental.pallas import tpu as pltpu
from jax.experimental.pallas import tpu_sc as plsc

N = 10000
E = 320000
D = 128
DE = 16
H = 64
MAX_NEIGHBORS = 32.0

_NTILES = 32
_C = 128
_NCHUNK = E // _C
_ROWS_PT = 632
_NPAD = 16 * _ROWS_PT


def _ssp(x):
    return jax.nn.softplus(x) - jnp.log(2.0)


_LOG2E = 1.4426950408889634
_LN2 = 0.6931471805599453


def _ssp_fast(x):
    a = x * _LOG2E
    t = jnp.exp2(-jnp.abs(a))
    return (jnp.maximum(a, 0.0) + jnp.log2(1.0 + t) - 1.0) * _LN2


def _node_proj_body(x_ref, w_ref, b_ref, o_ref):
    o_ref[...] = (
        jnp.dot(x_ref[...], w_ref[...], preferred_element_type=jnp.float32)
        + b_ref[...]
    )


def _node_proj(x, w, b):
    bn2 = 1000
    wbd = jnp.kron(jnp.eye(2, dtype=jnp.float32), w)
    bbd = jnp.tile(b, 2).reshape(1, 2 * H)
    return pl.pallas_call(
        _node_proj_body,
        grid=(N // 2 // bn2,),
        in_specs=[
            pl.BlockSpec((bn2, 2 * D), lambda i: (i, 0)),
            pl.BlockSpec((2 * D, 2 * H), lambda i: (0, 0)),
            pl.BlockSpec((1, 2 * H), lambda i: (0, 0)),
        ],
        out_specs=pl.BlockSpec((bn2, 2 * H), lambda i: (i, 0)),
        out_shape=jax.ShapeDtypeStruct((N // 2, 2 * H), jnp.float32),
    )(x.reshape(N // 2, 2 * D), wbd, bbd)


_E2 = E // 2


def _edge_mlp_half(eft, w1, b1, w2, b2):
    z1 = lax.dot_general(
        eft, w1, (((0,), (0,)), ((), ())),
        preferred_element_type=jnp.float32,
    ) + b1
    h1 = _ssp_fast(z1)
    z2 = jnp.dot(h1, w2, preferred_element_type=jnp.float32) + b2
    return _ssp_fast(z2)


def _edge_mlp_body(efta_ref, eftb_ref, w1_ref, b1_ref, w2_ref, b2_ref, o_ref):
    w1, b1 = w1_ref[...], b1_ref[...]
    w2, b2 = w2_ref[...], b2_ref[...]
    hea = _edge_mlp_half(efta_ref[...], w1, b1, w2, b2)
    heb = _edge_mlp_half(eftb_ref[...], w1, b1, w2, b2)
    o_ref[...] = jnp.concatenate([hea, heb], axis=1)


_E4 = E // 4


def _edge_mlp(ef, w1, b1, w2, b2, part):
    be = 16000
    nblk = _E4 // be
    p0 = part * 2 * nblk
    eft = ef.T
    return pl.pallas_call(
        _edge_mlp_body,
        grid=(nblk,),
        in_specs=[
            pl.BlockSpec((DE, be), lambda i, p0=p0: (0, p0 + i)),
            pl.BlockSpec((DE, be), lambda i, p0=p0, nblk=nblk: (0, p0 + nblk + i)),
            pl.BlockSpec((DE, H), lambda i: (0, 0)),
            pl.BlockSpec((1, H), lambda i: (0, 0)),
            pl.BlockSpec((H, H), lambda i: (0, 0)),
            pl.BlockSpec((1, H), lambda i: (0, 0)),
        ],
        out_specs=pl.BlockSpec((be, 128), lambda i: (i, 0)),
        out_shape=jax.ShapeDtypeStruct((_E4, 128), jnp.float32),
        compiler_params=pltpu.CompilerParams(fuse_transposed_lhs_in_matmul=True),
    )(eft, eft, w1, b1.reshape(1, H), w2, b2.reshape(1, H))


def _final_body(pa_ref, wo_ref, bo_ref, wp_ref, bp_ref, o_ref):
    h = pa_ref[0] + pa_ref[1]
    h = _ssp_fast(
        jnp.dot(h, wo_ref[...], preferred_element_type=jnp.float32) + bo_ref[...]
    ) * (1.0 / MAX_NEIGHBORS)
    o_ref[...] = (
        jnp.dot(h, wp_ref[...], preferred_element_type=jnp.float32)
        + bp_ref[...]
    )


def _final(pa, wo, bo, wp, bp):
    bn = 2000
    return pl.pallas_call(
        _final_body,
        grid=(N // bn,),
        in_specs=[
            pl.BlockSpec((2, bn, H), lambda i: (0, i, 0)),
            pl.BlockSpec((H, D), lambda i: (0, 0)),
            pl.BlockSpec((1, D), lambda i: (0, 0)),
            pl.BlockSpec((D, D), lambda i: (0, 0)),
            pl.BlockSpec((1, D), lambda i: (0, 0)),
        ],
        out_specs=pl.BlockSpec((bn, D), lambda i: (i, 0)),
        out_shape=jax.ShapeDtypeStruct((N, D), jnp.float32),
    )(pa, wo, bo.reshape(1, D), wp, bp.reshape(1, D))


_SLOTS = 40
_NCH = _NCHUNK // 2


def _sc_aggregate(hv, he, src2d, dst2d, zeros, part):
    mesh = plsc.VectorSubcoreMesh(core_axis_name="c", subcore_axis_name="s")

    @functools.partial(
        pl.kernel,
        mesh=mesh,
        out_type=jax.ShapeDtypeStruct((2 * _NPAD, H), jnp.float32),
        scratch_types=[
            pltpu.VMEM((_SLOTS, _C), jnp.int32),
            pltpu.VMEM((_SLOTS, _C), jnp.int32),
            pltpu.VMEM((_C, H), jnp.float32),
            pltpu.VMEM((_C, H), jnp.float32),
            pltpu.VMEM((_C, H), jnp.float32),
            pltpu.VMEM((_C, H), jnp.float32),
            pltpu.VMEM_SHARED((_NPAD, H), jnp.float32),
            pltpu.SemaphoreType.DMA,
            pltpu.SemaphoreType.DMA,
            pltpu.SemaphoreType.DMA,
            pltpu.SemaphoreType.DMA,
        ],
        compiler_params=pltpu.CompilerParams(use_tc_tiling_on_sc=False),
    )
    def body(hv_hbm, he_hbm, src_hbm, dst_hbm, z_hbm, out_hbm,
             src_all, dst_all, rows0, rows1, he0, he1, acc,
             sg0, sg1, sh0, sh1):
        cid = lax.axis_index("c")
        sid = lax.axis_index("s")
        g = cid * 16 + sid
        base = part * _NCH
        r0 = base + g * 39 + jnp.minimum(g, 2)
        nreal = 39 + jnp.where(g < 2, 1, 0)
        wstart = jnp.minimum(r0, base + _NCH - _SLOTS)
        off = r0 - wstart

        rows_b = (rows0, rows1)
        he_b = (he0, he1)
        sg_b = (sg0, sg1)
        sh_b = (sh0, sh1)

        pltpu.sync_copy(src_hbm.at[pl.ds(wstart, _SLOTS)], src_all)
        pltpu.sync_copy(dst_hbm.at[pl.ds(wstart, _SLOTS)], dst_all)

        def idxrow(i):
            return jnp.minimum(off + i, _SLOTS - 1)

        def issue(i, b):
            row_cl = jnp.minimum(r0 + i, base + _NCH - 1) - base
            rsel = jnp.where(row_cl < _NCH // 2,
                             row_cl, row_cl - _NCH // 2) * _C
            csel = jnp.where(row_cl < _NCH // 2, 0, H)
            pltpu.async_copy(
                he_hbm.at[pl.ds(rsel, _C), pl.ds(csel, H)], he_b[b], sh_b[b])
            pltpu.async_copy(hv_hbm.at[src_all.at[idxrow(i)]], rows_b[b], sg_b[b])

        issue(0, 0)
        issue(1, 1)

        pltpu.sync_copy(
            z_hbm.at[pl.ds(cid * _NPAD + sid * _ROWS_PT, _ROWS_PT)],
            acc.at[pl.ds(sid * _ROWS_PT, _ROWS_PT)],
        )
        plsc.subcore_barrier()

        def step(io, carry):
            for b in range(2):
                i = io * 2 + b
                pltpu.make_async_copy(
                    he_hbm.at[pl.ds(0, _C), pl.ds(0, H)], he_b[b], sh_b[b]).wait()
                pltpu.make_async_copy(
                    hv_hbm.at[src_all.at[0]], rows_b[b], sg_b[b]).wait()

                rv, hv_ = rows_b[b], he_b[b]

                def mul2(q, c2, rv=rv, hv_=hv_):
                    for rr in range(2):
                        r = q * 2 + rr
                        for j in range(H // 16):
                            sl = pl.ds(j * 16, 16)
                            rv[r, sl] = rv[r, sl] * hv_[r, sl]
                    return c2

                lax.fori_loop(0, _C // 2, mul2, 0)

                @pl.when(i < nreal)
                def _(b=b, i=i):
                    pltpu.sync_copy(rows_b[b], acc.at[dst_all.at[idxrow(i)]],
                                    add=True)

                @pl.when(i + 2 < _SLOTS)
                def _(b=b, i=i):
                    issue(i + 2, b)
            return carry

        lax.fori_loop(0, _SLOTS // 2, step, 0)
        plsc.subcore_barrier()

        pltpu.sync_copy(
            acc.at[pl.ds(sid * _ROWS_PT, _ROWS_PT)],
            out_hbm.at[pl.ds(cid * _NPAD + sid * _ROWS_PT, _ROWS_PT)],
        )

    return body(hv, he, src2d, dst2d, zeros)


def kernel(node_feats, edge_feats, edge_index, Wn, bn, We1, be1, We2, be2,
           Wo, bo, Wp, bp):
    src2d = edge_index[0].reshape(_NCHUNK, _C)
    dst2d = edge_index[1].reshape(_NCHUNK, _C)
    hv = _node_proj(node_feats, Wn, bn).reshape(N, H)
    zeros = jnp.zeros((2 * _NPAD, H), jnp.float32)
    he0 = _edge_mlp(edge_feats, We1, be1, We2, be2, 0)
    p0 = _sc_aggregate(hv, he0, src2d, dst2d, zeros, 0)
    he1 = _edge_mlp(edge_feats, We1, be1, We2, be2, 1)
    p1 = _sc_aggregate(hv, he1, src2d, dst2d, p0, 1)
    return _final(p1.reshape(2, _NPAD, H), Wo, bo, Wp, bp)
